# Initial kernel scaffold; baseline (speedup 1.0000x reference)
import jax
import jax.numpy as jnp
from jax.experimental import pallas as pl


def kernel(x, edge_index, y, batch, edge_weight_param, W0a, W1a, ba, W0b, W1b, bb, fcW, fcb):
    raise NotImplementedError("write your pallas kernel here")


# speedup vs baseline: 575.4864x; 575.4864x over previous
import jax
import jax.numpy as jnp
from jax.experimental import pallas as pl

N_NODES = 62
BATCH = 256
FEAT = 64
NPAD = 64


def _eeg_kernel(xp_ref, p_ref, w0at_ref, w1at_ref, w0bt_ref, w1bt_ref,
                bat_ref, bb_ref, fcw_ref, out_ref):
    pv = p_ref[:, :]
    rows = []
    for r in range(N_NODES):
        off = r * (r + 1) // 2
        rows.append(jax.lax.slice(pv, (0, off), (1, off + NPAD)))
    rows.append(jnp.zeros((NPAD - N_NODES, NPAD), jnp.float32))
    P = jnp.concatenate(rows, axis=0)
    row_id = jax.lax.broadcasted_iota(jnp.int32, (NPAD, NPAD), 0)
    col_id = jax.lax.broadcasted_iota(jnp.int32, (NPAD, NPAD), 1)
    P = jnp.where((col_id <= row_id) & (row_id < N_NODES), P, 0.0)
    A = jnp.where(col_id != row_id, P + P.T, 0.0)
    deg_c = jnp.sum(A, axis=1, keepdims=True)
    deg_r = jnp.sum(A, axis=0, keepdims=True)
    dis_c = jnp.where(deg_c > 0, 1.0 / jnp.sqrt(jnp.maximum(deg_c, 1e-12)), 0.0)
    dis_r = jnp.where(deg_r > 0, 1.0 / jnp.sqrt(jnp.maximum(deg_r, 1e-12)), 0.0)
    S = -(dis_c * A * dis_r)
    srow = jnp.sum(S, axis=0, keepdims=True)

    w0at = w0at_ref[:, :]
    w1at = w1at_ref[:, :]
    w0bt = w0bt_ref[:, :]
    w1bt = w1bt_ref[:, :]
    m0 = jnp.dot(w0bt, w0at)
    m1 = jnp.dot(w1bt, w0at) + jnp.dot(w0bt, w1at)
    m2 = jnp.dot(w1bt, w1at)
    bat = bat_ref[:, :]
    alpha = jnp.sum(bat * w0bt, axis=1, keepdims=True) + bb_ref[:, :]
    beta = jnp.sum(bat * w1bt, axis=1, keepdims=True)

    xp = xp_ref[:, :, :]
    E0 = jnp.sum(xp * m0.reshape(1, 1, NPAD), axis=-1)
    E1 = jnp.sum(xp * m1.reshape(1, 1, NPAD), axis=-1)
    E2 = jnp.sum(xp * m2.reshape(1, 1, NPAD), axis=-1)

    T = E1 + jnp.dot(E2, S)
    H = E0 + jnp.dot(T, S) + alpha + beta * srow
    out_ref[:, :] = jnp.dot(H, fcw_ref[:, :])


def kernel(x, edge_index, y, batch, edge_weight_param, W0a, W1a, ba,
           W0b, W1b, bb, fcW, fcb):
    bsz = y.shape[0]
    xp = jnp.pad(x.reshape(bsz, N_NODES, FEAT),
                 ((0, 0), (0, NPAD - N_NODES), (0, 0)))
    p_pad = jnp.zeros((1, 2048), jnp.float32).at[0, :edge_weight_param.shape[0]].set(edge_weight_param)
    w0at = W0a.T
    w1at = W1a.T
    w0bt = W0b.T
    w1bt = W1b.T
    bat = ba.reshape(1, FEAT)
    bbs = bb.reshape(1, 1)
    fcw = jnp.zeros((NPAD, 128), jnp.float32).at[:N_NODES, :fcW.shape[0]].set(fcW.T)

    out = pl.pallas_call(
        _eeg_kernel,
        out_shape=jax.ShapeDtypeStruct((bsz, 128), jnp.float32),
    )(xp, p_pad, w0at, w1at, w0bt, w1bt, bat, bbs, fcw)
    return out[:, :fcW.shape[0]] + fcb


# speedup vs baseline: 1146.0159x; 1.9914x over previous
import jax
import jax.numpy as jnp
from jax.experimental import pallas as pl

N_NODES = 62
BATCH = 256
FEAT = 64
NPAD = 64
HP = jax.lax.Precision.HIGHEST


def _eeg_kernel(xr_ref, p_ref, w0at_ref, w1at_ref, w0bt_ref, w1bt_ref,
                bat_ref, bb_ref, fcw_ref, fcb_ref, out_ref):
    pv = p_ref[:, :]
    rows = []
    for r in range(N_NODES):
        off = r * (r + 1) // 2
        rows.append(jax.lax.slice(pv, (0, off), (1, off + NPAD)))
    rows.append(jnp.zeros((NPAD - N_NODES, NPAD), jnp.float32))
    P = jnp.concatenate(rows, axis=0)
    row_id = jax.lax.broadcasted_iota(jnp.int32, (NPAD, NPAD), 0)
    col_id = jax.lax.broadcasted_iota(jnp.int32, (NPAD, NPAD), 1)
    P = jnp.where((col_id <= row_id) & (row_id < N_NODES), P, 0.0)
    A = jnp.where(col_id != row_id, P + P.T, 0.0)
    deg_c = jnp.sum(A, axis=1, keepdims=True)
    deg_r = jnp.sum(A, axis=0, keepdims=True)
    dis_c = jnp.where(deg_c > 0, 1.0 / jnp.sqrt(jnp.maximum(deg_c, 1e-12)), 0.0)
    dis_r = jnp.where(deg_r > 0, 1.0 / jnp.sqrt(jnp.maximum(deg_r, 1e-12)), 0.0)
    S = -(dis_c * A * dis_r)
    srow = jnp.sum(S, axis=0, keepdims=True)

    w0at = w0at_ref[:, :]
    w1at = w1at_ref[:, :]
    w0bt = w0bt_ref[:, :]
    w1bt = w1bt_ref[:, :]
    m0 = jnp.dot(w0bt, w0at, precision=HP)
    m1 = jnp.dot(w1bt, w0at, precision=HP) + jnp.dot(w0bt, w1at, precision=HP)
    m2 = jnp.dot(w1bt, w1at, precision=HP)
    bat = bat_ref[:, :]
    alpha = jnp.sum(bat * w0bt, axis=1, keepdims=True) + bb_ref[:, :]
    beta = jnp.sum(bat * w1bt, axis=1, keepdims=True)

    R0 = fcw_ref[:, :]
    R1 = jnp.dot(S, R0, precision=HP)
    R2 = jnp.dot(S, R1, precision=HP)

    q = (m0.reshape(1, NPAD, 1) * jax.lax.slice(R0, (0, 0), (N_NODES, 128)).reshape(N_NODES, 1, 128)
         + m1.reshape(1, NPAD, 1) * jax.lax.slice(R1, (0, 0), (N_NODES, 128)).reshape(N_NODES, 1, 128)
         + m2.reshape(1, NPAD, 1) * jax.lax.slice(R2, (0, 0), (N_NODES, 128)).reshape(N_NODES, 1, 128))
    Q = q.reshape(N_NODES * FEAT, 128)

    bias = jnp.dot(alpha + beta * srow, R0, precision=HP)
    out_ref[:, :] = (jnp.dot(xr_ref[:, :], Q, precision=HP)
                     + bias + fcb_ref[:, :])


def kernel(x, edge_index, y, batch, edge_weight_param, W0a, W1a, ba,
           W0b, W1b, bb, fcW, fcb):
    bsz = y.shape[0]
    xr = x.reshape(bsz, N_NODES * FEAT)
    p_pad = jnp.zeros((1, 2048), jnp.float32).at[0, :edge_weight_param.shape[0]].set(edge_weight_param)
    w0at = W0a.T
    w1at = W1a.T
    w0bt = W0b.T
    w1bt = W1b.T
    bat = ba.reshape(1, FEAT)
    bbs = bb.reshape(1, 1)
    fcw = jnp.zeros((NPAD, 128), jnp.float32).at[:N_NODES, :fcW.shape[0]].set(fcW.T)
    fcbp = jnp.zeros((1, 128), jnp.float32).at[0, :fcb.shape[0]].set(fcb)

    out = pl.pallas_call(
        _eeg_kernel,
        out_shape=jax.ShapeDtypeStruct((bsz, 128), jnp.float32),
    )(xr, p_pad, w0at, w1at, w0bt, w1bt, bat, bbs, fcw, fcbp)
    return out[:, :fcW.shape[0]]


# speedup vs baseline: 1346.2700x; 1.1747x over previous
import jax
import jax.numpy as jnp
from jax.experimental import pallas as pl

N_NODES = 62
FEAT = 64
NPAD = 64
N_TRIL = N_NODES * (N_NODES + 1) // 2
HP = jax.lax.Precision.HIGHEST


def _eeg_kernel(xr_ref, p_ref, w0a_ref, w1a_ref, w0b_ref, w1b_ref,
                ba_ref, bb_ref, fcw_ref, fcb_ref, out_ref):
    pv = p_ref[:, :]
    rows = []
    for r in range(N_NODES):
        off = r * (r + 1) // 2
        rows.append(jax.lax.slice(pv, (0, off), (1, off + N_NODES)))
    P62 = jnp.concatenate(rows, axis=0)
    P = jnp.concatenate(
        [jnp.concatenate([P62, jnp.zeros((N_NODES, NPAD - N_NODES), jnp.float32)], axis=1),
         jnp.zeros((NPAD - N_NODES, NPAD), jnp.float32)], axis=0)
    row_id = jax.lax.broadcasted_iota(jnp.int32, (NPAD, NPAD), 0)
    col_id = jax.lax.broadcasted_iota(jnp.int32, (NPAD, NPAD), 1)
    A = jnp.where((col_id < row_id) & (row_id < N_NODES), P, 0.0)
    A = A + A.T
    deg_c = jnp.sum(A, axis=1, keepdims=True)
    deg_r = jnp.sum(A, axis=0, keepdims=True)
    dis_c = jnp.where(deg_c > 0, 1.0 / jnp.sqrt(jnp.maximum(deg_c, 1e-12)), 0.0)
    dis_r = jnp.where(deg_r > 0, 1.0 / jnp.sqrt(jnp.maximum(deg_r, 1e-12)), 0.0)
    S = -(dis_c * A * dis_r)
    srow = jnp.sum(S, axis=0, keepdims=True)

    w0a = w0a_ref[:, :]
    w1a = w1a_ref[:, :]
    w0b = w0b_ref[:, :]
    w1b = w1b_ref[:, :]
    m0 = jnp.dot(w0a, w0b, precision=HP)
    m1 = jnp.dot(w0a, w1b, precision=HP) + jnp.dot(w1a, w0b, precision=HP)
    m2 = jnp.dot(w1a, w1b, precision=HP)
    alpha = jnp.dot(ba_ref[:, :], w0b, precision=HP) + bb_ref[:, :]
    beta = jnp.dot(ba_ref[:, :], w1b, precision=HP)

    R0 = jnp.concatenate(
        [jnp.transpose(fcw_ref[:, :]),
         jnp.zeros((NPAD - N_NODES, 3), jnp.float32)], axis=0)
    R1 = jnp.dot(S, R0, precision=HP)
    R2 = jnp.dot(S, R1, precision=HP)

    q = (m0.reshape(1, FEAT, 1) * jax.lax.slice(R0, (0, 0), (N_NODES, 3)).reshape(N_NODES, 1, 3)
         + m1.reshape(1, FEAT, 1) * jax.lax.slice(R1, (0, 0), (N_NODES, 3)).reshape(N_NODES, 1, 3)
         + m2.reshape(1, FEAT, 1) * jax.lax.slice(R2, (0, 0), (N_NODES, 3)).reshape(N_NODES, 1, 3))
    Q = q.reshape(N_NODES * FEAT, 3)

    bias = jnp.dot(alpha + beta * srow, R0, precision=HP)
    out_ref[:, :] = (jnp.dot(xr_ref[:, :], Q, precision=HP)
                     + bias + fcb_ref[:, :])


def kernel(x, edge_index, y, batch, edge_weight_param, W0a, W1a, ba,
           W0b, W1b, bb, fcW, fcb):
    bsz = y.shape[0]
    xr = x.reshape(bsz, N_NODES * FEAT)
    p2 = edge_weight_param.reshape(1, N_TRIL)
    ba_r = ba.reshape(1, FEAT)
    bb_r = bb.reshape(1, 1)
    fcb_r = fcb.reshape(1, 3)

    return pl.pallas_call(
        _eeg_kernel,
        out_shape=jax.ShapeDtypeStruct((bsz, 3), jnp.float32),
    )(xr, p2, W0a, W1a, W0b, W1b, ba_r, bb_r, fcW, fcb_r)
